# baseline (speedup 1.0000x reference)
import dataclasses
import functools

import jax
import jax.numpy as jnp
import numpy as np
from jax import lax
from jax.experimental import pallas as pl
from jax.experimental.pallas import tpu as pltpu
from jax.experimental.pallas import tpu_sc as plsc

N = 2048
CUT2 = 25.0
EPS2 = 1e-10
MAX_EDGES = 120000
NW = 32
ROWS_W = N // NW
LANES = 16
CAP_ROWS = 96
CAP = CAP_ROWS * 128
TRASH = MAX_EDGES + 448
PAD_TOTAL = 120832
SLICE_W = PAD_TOTAL // NW
BLKR = 8
NBLK = ROWS_W // BLKR


def _kmap_body(fr_ref, ft_ref, cr_ref, ct_ref, sc_ref, kmap_ref, cnt_ref):
  fr = fr_ref[...]
  sct = sc_ref[...]

  frel0 = ft_ref[0:1, :] - fr[:, 0:1]
  frel1 = ft_ref[1:2, :] - fr[:, 1:2]
  frel2 = ft_ref[2:3, :] - fr[:, 2:3]
  s0 = jnp.round(frel0)
  s1 = jnp.round(frel1)
  s2 = jnp.round(frel2)
  kf = (1.0 - s0) * 9.0 + (1.0 - s1) * 3.0 + (1.0 - s2)
  ki = kf.astype(jnp.int32)

  cr = cr_ref[...]
  b0 = ct_ref[0:1, :] - cr[:, 0:1]
  b1 = ct_ref[1:2, :] - cr[:, 1:2]
  b2 = ct_ref[2:3, :] - cr[:, 2:3]

  sc0 = jnp.zeros_like(b0)
  sc1 = jnp.zeros_like(b1)
  sc2 = jnp.zeros_like(b2)
  for k in range(27):
    m = ki == k
    sc0 = jnp.where(m, sct[k, 0], sc0)
    sc1 = jnp.where(m, sct[k, 1], sc1)
    sc2 = jnp.where(m, sct[k, 2], sc2)

  v0 = b0 + sc0
  v1 = b1 + sc1
  v2 = b2 + sc2
  q0 = v0 * v0
  q1 = v1 * v1
  q2 = v2 * v2
  d2 = (q0 + q1) + q2
  edge = (d2 <= CUT2) & (d2 > EPS2)
  kmap_ref[...] = jnp.where(edge, ki, -1)

  kiota = lax.broadcasted_iota(jnp.int32, (1, 1, 32), 2)
  acc = jnp.zeros((1, 1, 32), jnp.int32)
  for k in range(27):
    ck = jnp.sum(jnp.where(edge & (ki == k), 1, 0))
    acc = jnp.where(kiota == k, ck, acc)
  cnt_ref[...] = acc


def _kmap_call(frac, frac_t, cart, cart_t, shift_cart):
  grid = N // ROWS_W
  return pl.pallas_call(
      _kmap_body,
      grid=(grid,),
      in_specs=[
          pl.BlockSpec((ROWS_W, 3), lambda b: (b, 0)),
          pl.BlockSpec((3, N), lambda b: (0, 0)),
          pl.BlockSpec((ROWS_W, 3), lambda b: (b, 0)),
          pl.BlockSpec((3, N), lambda b: (0, 0)),
          pl.BlockSpec((27, 3), lambda b: (0, 0)),
      ],
      out_specs=[
          pl.BlockSpec((ROWS_W, N), lambda b: (b, 0)),
          pl.BlockSpec((1, 1, 32), lambda b: (b, 0, 0)),
      ],
      out_shape=[
          jax.ShapeDtypeStruct((N, N), jnp.int32),
          jax.ShapeDtypeStruct((NW, 1, 32), jnp.int32),
      ],
  )(frac, frac_t, cart, cart_t, shift_cart)


def _incl_cumsum_rows(c, n):
  sh = 1
  while sh < n:
    z = jnp.zeros((sh, c.shape[1]), c.dtype)
    c = c + jnp.concatenate([z, c[: n - sh, :]], axis=0)
    sh *= 2
  return c


def _incl_cumsum_lanes(c, n):
  sh = 1
  while sh < n:
    z = jnp.zeros((c.shape[0], sh), c.dtype)
    c = c + jnp.concatenate([z, c[:, : n - sh]], axis=1)
    sh *= 2
  return c


def _offsets_body(cnt_ref, offs_ref, nev_ref):
  c = cnt_ref[...].reshape(NW, 32)
  colsum = jnp.sum(c, axis=0).reshape(1, 32)
  prefk = _incl_cumsum_lanes(colsum, 32) - colsum
  rowp = _incl_cumsum_rows(c, NW) - c
  offs_ref[...] = prefk + rowp
  total = jnp.sum(colsum)
  nev_ref[...] = jnp.full((8, 128), total, jnp.int32)


def _offsets_call(counts):
  return pl.pallas_call(
      _offsets_body,
      out_shape=[
          jax.ShapeDtypeStruct((NW, 32), jnp.int32),
          jax.ShapeDtypeStruct((8, 128), jnp.int32),
      ],
  )(counts)


def _sc_mesh():
  return plsc.VectorSubcoreMesh(core_axis_name="c", subcore_axis_name="s")


def _sc_params():
  cp = pltpu.CompilerParams()
  if "needs_layout_passes" in pltpu.CompilerParams.__dataclass_fields__:
    cp = dataclasses.replace(cp, needs_layout_passes=False)
  return cp


def _scatter_kernel(kmap_hbm, offs_hbm, nev_hbm, packed_hbm,
                    rowbuf, rowbuf2, cur, nevbuf, didx, lpack,
                    sem0, sem1, dsem):
  wid = lax.axis_index("s") * 2 + lax.axis_index("c")
  base_row = wid * ROWS_W
  iota = lax.iota(jnp.int32, LANES)
  ones = jnp.ones((LANES,), jnp.int32)
  zeros = jnp.zeros((LANES,), jnp.int32)

  pltpu.sync_copy(offs_hbm.at[pl.ds(wid * 32, 32)], cur)
  pltpu.sync_copy(nev_hbm.at[pl.ds(0, LANES)], nevbuf)
  nev_vec = nevbuf[...]

  trash = jnp.full((LANES,), TRASH, jnp.int32)

  @pl.loop(0, CAP_ROWS)
  def _init(r):
    @pl.loop(0, 128 // LANES)
    def _init2(q):
      didx[r, pl.ds(q * LANES, LANES)] = trash

  def append(n, dest, values, mask):
    csm = jnp.cumsum(jnp.where(mask, 1, 0))
    pos = n + csm - 1
    mask = mask & (pos < CAP)
    pr = lax.shift_right_logical(pos, 7)
    pc = pos & 127
    destw = jnp.where(dest < MAX_EDGES, dest, TRASH)
    plsc.store_scatter(didx.at[:], [pr, pc], destw, mask=mask)
    plsc.store_scatter(lpack.at[:], [pr, pc], values, mask=mask)
    return n + jnp.sum(jnp.where(mask, 1, 0))

  def blk_dma(ib, buf, sem):
    return pltpu.async_copy(
        kmap_hbm.at[pl.ds((base_row + ib * BLKR) * N, BLKR * N)], buf, sem)

  def blk_wait(ib, buf, sem):
    pltpu.make_async_copy(
        kmap_hbm.at[pl.ds((base_row + ib * BLKR) * N, BLKR * N)], buf,
        sem).wait()

  def process(rbuf, blk_row, n):
    def rb(r, n):
      i_g = blk_row + r

      def vec_body(jv, n):
        kvec = rbuf[pl.ds(r * N + jv * LANES, LANES)]
        mask = kvec >= 0

        def do(n):
          kcl = kvec & 31
          r_in, lastm = plsc.scan_count(kcl, mask)
          rr = r_in - 1
          curv = plsc.load_gather(cur.at[:], [kcl])
          dest = curv + rr
          packed = (kcl << 22) | (i_g << 11) | (jv * LANES + iota)
          n2 = append(n, dest, packed, mask)
          plsc.store_scatter(cur.at[:], [kcl], dest + ones,
                             mask=mask & lastm)
          return n2

        return lax.cond(jnp.any(mask), do, lambda n: n, n)

      return lax.fori_loop(0, N // LANES, vec_body, n)

    return lax.fori_loop(0, BLKR, rb, n)

  blk_dma(0, rowbuf, sem0)

  def pair_body(t, n):
    b0 = 2 * t
    blk_wait(b0, rowbuf, sem0)
    blk_dma(b0 + 1, rowbuf2, sem1)
    n = process(rowbuf, base_row + b0 * BLKR, n)

    def prefetch(x):
      blk_dma(b0 + 2, rowbuf, sem0)
      return x

    lax.cond(t < NBLK // 2 - 1, prefetch, lambda x: x, 0)
    blk_wait(b0 + 1, rowbuf2, sem1)
    return process(rowbuf2, base_row + (b0 + 1) * BLKR, n)

  n = lax.fori_loop(0, NBLK // 2, pair_body, jnp.int32(0))

  nch = lax.shift_right_logical(n + 127, 7)

  def fire(ci, x):
    pltpu.async_copy(lpack.at[ci], packed_hbm.at[didx.at[ci]], dsem)
    return x

  lax.fori_loop(0, nch, fire, 0)

  def drain(ci, x):
    pltpu.make_async_copy(lpack.at[ci], packed_hbm.at[didx.at[ci]],
                          dsem).wait()
    return x

  lax.fori_loop(0, nch, drain, 0)


def _scatter_call(kmap, offs, nev):
  kern = pl.kernel(
      _scatter_kernel,
      out_type=jax.ShapeDtypeStruct((PAD_TOTAL,), jnp.int32),
      mesh=_sc_mesh(),
      compiler_params=_sc_params(),
      scratch_types=[
          pltpu.VMEM((BLKR * N,), jnp.int32),
          pltpu.VMEM((BLKR * N,), jnp.int32),
          pltpu.VMEM((32,), jnp.int32),
          pltpu.VMEM((LANES,), jnp.int32),
          pltpu.VMEM((CAP_ROWS, 128), jnp.int32),
          pltpu.VMEM((CAP_ROWS, 128), jnp.int32),
          pltpu.SemaphoreType.DMA,
          pltpu.SemaphoreType.DMA,
          pltpu.SemaphoreType.DMA,
      ],
  )
  return kern(kmap, offs, nev)


def _expand_kernel(packed_hbm, nev_hbm, cart_t_hbm, shift_t_hbm,
                   src_hbm, dst_hbm, sid_hbm, vx_hbm, vy_hbm, vz_hbm,
                   pbuf, nevbuf, osrc, odst, osid, ovx, ovy, ovz,
                   cx, cy, cz, scx, scy, scz):
  wid = lax.axis_index("s") * 2 + lax.axis_index("c")
  base = wid * SLICE_W
  pltpu.sync_copy(packed_hbm.at[pl.ds(base, SLICE_W)], pbuf)
  pltpu.sync_copy(nev_hbm.at[pl.ds(0, LANES)], nevbuf)
  pltpu.sync_copy(cart_t_hbm.at[pl.ds(0, N)], cx)
  pltpu.sync_copy(cart_t_hbm.at[pl.ds(N, N)], cy)
  pltpu.sync_copy(cart_t_hbm.at[pl.ds(2 * N, N)], cz)
  pltpu.sync_copy(shift_t_hbm.at[pl.ds(0, 32)], scx)
  pltpu.sync_copy(shift_t_hbm.at[pl.ds(32, 32)], scy)
  pltpu.sync_copy(shift_t_hbm.at[pl.ds(64, 32)], scz)

  nev_vec = nevbuf[...]
  iota = lax.iota(jnp.int32, LANES)

  @pl.loop(0, SLICE_W // LANES)
  def _(v):
    sl = pl.ds(v * LANES, LANES)
    gpos = base + v * LANES + iota
    p = jnp.where(gpos < nev_vec, pbuf[sl], 0)
    k = lax.shift_right_logical(p, 22) & 31
    i = lax.shift_right_logical(p, 11) & (N - 1)
    j = p & (N - 1)
    osrc[sl] = i
    odst[sl] = j
    osid[sl] = k
    ovx[sl] = (plsc.load_gather(cx.at[:], [j])
               - plsc.load_gather(cx.at[:], [i])) + plsc.load_gather(
                   scx.at[:], [k])
    ovy[sl] = (plsc.load_gather(cy.at[:], [j])
               - plsc.load_gather(cy.at[:], [i])) + plsc.load_gather(
                   scy.at[:], [k])
    ovz[sl] = (plsc.load_gather(cz.at[:], [j])
               - plsc.load_gather(cz.at[:], [i])) + plsc.load_gather(
                   scz.at[:], [k])

  out_sl = pl.ds(base, SLICE_W)
  pltpu.sync_copy(osrc, src_hbm.at[out_sl])
  pltpu.sync_copy(odst, dst_hbm.at[out_sl])
  pltpu.sync_copy(osid, sid_hbm.at[out_sl])
  pltpu.sync_copy(ovx, vx_hbm.at[out_sl])
  pltpu.sync_copy(ovy, vy_hbm.at[out_sl])
  pltpu.sync_copy(ovz, vz_hbm.at[out_sl])


def _expand_call(packed, nev, cart_t, shift_t):
  kern = pl.kernel(
      _expand_kernel,
      out_type=[jax.ShapeDtypeStruct((PAD_TOTAL,), jnp.int32)] * 3
      + [jax.ShapeDtypeStruct((PAD_TOTAL,), jnp.float32)] * 3,
      mesh=_sc_mesh(),
      compiler_params=_sc_params(),
      scratch_types=[
          pltpu.VMEM((SLICE_W,), jnp.int32),
          pltpu.VMEM((LANES,), jnp.int32),
          pltpu.VMEM((SLICE_W,), jnp.int32),
          pltpu.VMEM((SLICE_W,), jnp.int32),
          pltpu.VMEM((SLICE_W,), jnp.int32),
          pltpu.VMEM((SLICE_W,), jnp.float32),
          pltpu.VMEM((SLICE_W,), jnp.float32),
          pltpu.VMEM((SLICE_W,), jnp.float32),
          pltpu.VMEM((N,), jnp.float32),
          pltpu.VMEM((N,), jnp.float32),
          pltpu.VMEM((N,), jnp.float32),
          pltpu.VMEM((32,), jnp.float32),
          pltpu.VMEM((32,), jnp.float32),
          pltpu.VMEM((32,), jnp.float32),
      ],
  )
  return kern(packed, nev, cart_t, shift_t)


def _dist_body(vx_ref, vy_ref, vz_ref, d_ref):
  vx = vx_ref[...]
  vy = vy_ref[...]
  vz = vz_ref[...]
  q = (vx * vx + vy * vy) + vz * vz
  d_ref[...] = jnp.sqrt(q + 1e-12)


def _dist_call(vx, vy, vz):
  return pl.pallas_call(
      _dist_body,
      out_shape=jax.ShapeDtypeStruct((8, PAD_TOTAL // 8), jnp.float32),
  )(vx, vy, vz)


def kernel(frac_coords, lattice):
  frac = frac_coords.astype(jnp.float32)
  cart = frac @ lattice
  g = np.array([-1.0, 0.0, 1.0])
  shifts = np.stack(np.meshgrid(g, g, g, indexing="ij"), axis=-1).reshape(-1, 3)
  shifts = jnp.asarray(shifts, dtype=jnp.float32)
  shift_cart = shifts @ lattice

  frac_t = frac.T
  cart_t = cart.T
  shift_pad = jnp.concatenate(
      [shift_cart, jnp.zeros((5, 3), jnp.float32)], axis=0)
  shift_t = shift_pad.T

  kmap, counts = _kmap_call(frac, frac_t, cart, cart_t, shift_cart)
  offs, nev = _offsets_call(counts)
  packed = _scatter_call(kmap.reshape(-1), offs.reshape(-1), nev.reshape(-1))
  src, dst, sidx, vx, vy, vz = _expand_call(packed, nev.reshape(-1),
                                            cart_t.reshape(-1),
                                            shift_t.reshape(-1))
  dist = _dist_call(vx.reshape(8, -1), vy.reshape(8, -1),
                    vz.reshape(8, -1)).reshape(-1)
  vec = jnp.stack([vx, vy, vz], axis=-1)
  n_edges = nev[0, 0]
  return (src[:MAX_EDGES], dst[:MAX_EDGES], vec[:MAX_EDGES],
          dist[:MAX_EDGES], n_edges)


# speedup vs baseline: 1.0933x; 1.0933x over previous
import dataclasses
import functools

import jax
import jax.numpy as jnp
import numpy as np
from jax import lax
from jax.experimental import pallas as pl
from jax.experimental.pallas import tpu as pltpu
from jax.experimental.pallas import tpu_sc as plsc

N = 2048
CUT2 = 25.0
EPS2 = 1e-10
MAX_EDGES = 120000
NW = 32
ROWS_W = N // NW
LANES = 16
CAP_ROWS = 96
CAP = CAP_ROWS * 128
TRASH = MAX_EDGES + 448
PAD_TOTAL = 120832
SLICE_W = PAD_TOTAL // NW
BLKR = 8
NBLK = ROWS_W // BLKR


def _kmap_body(fr_ref, ft_ref, cr_ref, ct_ref, sc_ref, kmap_ref, cnt_ref):
  fr = fr_ref[...]
  sct = sc_ref[...]

  frel0 = ft_ref[0:1, :] - fr[:, 0:1]
  frel1 = ft_ref[1:2, :] - fr[:, 1:2]
  frel2 = ft_ref[2:3, :] - fr[:, 2:3]
  s0 = jnp.round(frel0)
  s1 = jnp.round(frel1)
  s2 = jnp.round(frel2)
  kf = (1.0 - s0) * 9.0 + (1.0 - s1) * 3.0 + (1.0 - s2)
  ki = kf.astype(jnp.int32)

  cr = cr_ref[...]
  b0 = ct_ref[0:1, :] - cr[:, 0:1]
  b1 = ct_ref[1:2, :] - cr[:, 1:2]
  b2 = ct_ref[2:3, :] - cr[:, 2:3]

  sc0 = jnp.zeros_like(b0)
  sc1 = jnp.zeros_like(b1)
  sc2 = jnp.zeros_like(b2)
  for k in range(27):
    m = ki == k
    sc0 = jnp.where(m, sct[k, 0], sc0)
    sc1 = jnp.where(m, sct[k, 1], sc1)
    sc2 = jnp.where(m, sct[k, 2], sc2)

  v0 = b0 + sc0
  v1 = b1 + sc1
  v2 = b2 + sc2
  q0 = v0 * v0
  q1 = v1 * v1
  q2 = v2 * v2
  d2 = (q0 + q1) + q2
  edge = (d2 <= CUT2) & (d2 > EPS2)
  kmap_ref[...] = jnp.where(edge, ki, -1)

  kiota = lax.broadcasted_iota(jnp.int32, (1, 1, 32), 2)
  acc = jnp.zeros((1, 1, 32), jnp.int32)
  for k in range(27):
    ck = jnp.sum(jnp.where(edge & (ki == k), 1, 0))
    acc = jnp.where(kiota == k, ck, acc)
  cnt_ref[...] = acc


def _kmap_call(frac, frac_t, cart, cart_t, shift_cart):
  grid = N // ROWS_W
  return pl.pallas_call(
      _kmap_body,
      grid=(grid,),
      in_specs=[
          pl.BlockSpec((ROWS_W, 3), lambda b: (b, 0)),
          pl.BlockSpec((3, N), lambda b: (0, 0)),
          pl.BlockSpec((ROWS_W, 3), lambda b: (b, 0)),
          pl.BlockSpec((3, N), lambda b: (0, 0)),
          pl.BlockSpec((27, 3), lambda b: (0, 0)),
      ],
      out_specs=[
          pl.BlockSpec((ROWS_W, N), lambda b: (b, 0)),
          pl.BlockSpec((1, 1, 32), lambda b: (b, 0, 0)),
      ],
      out_shape=[
          jax.ShapeDtypeStruct((N, N), jnp.int32),
          jax.ShapeDtypeStruct((NW, 1, 32), jnp.int32),
      ],
  )(frac, frac_t, cart, cart_t, shift_cart)


def _incl_cumsum_rows(c, n):
  sh = 1
  while sh < n:
    z = jnp.zeros((sh, c.shape[1]), c.dtype)
    c = c + jnp.concatenate([z, c[: n - sh, :]], axis=0)
    sh *= 2
  return c


def _incl_cumsum_lanes(c, n):
  sh = 1
  while sh < n:
    z = jnp.zeros((c.shape[0], sh), c.dtype)
    c = c + jnp.concatenate([z, c[:, : n - sh]], axis=1)
    sh *= 2
  return c


def _offsets_body(cnt_ref, offs_ref, nev_ref):
  c = cnt_ref[...].reshape(NW, 32)
  colsum = jnp.sum(c, axis=0).reshape(1, 32)
  prefk = _incl_cumsum_lanes(colsum, 32) - colsum
  rowp = _incl_cumsum_rows(c, NW) - c
  offs_ref[...] = prefk + rowp
  total = jnp.sum(colsum)
  nev_ref[...] = jnp.full((8, 128), total, jnp.int32)


def _offsets_call(counts):
  return pl.pallas_call(
      _offsets_body,
      out_shape=[
          jax.ShapeDtypeStruct((NW, 32), jnp.int32),
          jax.ShapeDtypeStruct((8, 128), jnp.int32),
      ],
  )(counts)


def _sc_mesh():
  return plsc.VectorSubcoreMesh(core_axis_name="c", subcore_axis_name="s")


def _sc_params():
  cp = pltpu.CompilerParams()
  if "needs_layout_passes" in pltpu.CompilerParams.__dataclass_fields__:
    cp = dataclasses.replace(cp, needs_layout_passes=False)
  return cp


def _scatter_kernel(kmap_hbm, offs_hbm, nev_hbm, packed_hbm,
                    rowbuf, rowbuf2, cur, nevbuf, didx, lpack,
                    sem0, sem1, dsem):
  wid = lax.axis_index("s") * 2 + lax.axis_index("c")
  base_row = wid * ROWS_W
  iota = lax.iota(jnp.int32, LANES)
  ones = jnp.ones((LANES,), jnp.int32)
  zeros = jnp.zeros((LANES,), jnp.int32)

  pltpu.sync_copy(offs_hbm.at[pl.ds(wid * 32, 32)], cur)
  pltpu.sync_copy(nev_hbm.at[pl.ds(0, LANES)], nevbuf)
  nev_vec = nevbuf[...]

  trash = jnp.full((LANES,), TRASH, jnp.int32)

  @pl.loop(0, CAP_ROWS)
  def _init(r):
    @pl.loop(0, 128 // LANES)
    def _init2(q):
      didx[r, pl.ds(q * LANES, LANES)] = trash

  def append(n, dest, values, mask):
    csm = jnp.cumsum(jnp.where(mask, 1, 0))
    pos = n + csm - 1
    mask = mask & (pos < CAP)
    pr = lax.shift_right_logical(pos, 7)
    pc = pos & 127
    destw = jnp.where(dest < MAX_EDGES, dest, TRASH)
    plsc.store_scatter(didx.at[:], [pr, pc], destw, mask=mask)
    plsc.store_scatter(lpack.at[:], [pr, pc], values, mask=mask)
    return n + jnp.sum(jnp.where(mask, 1, 0))

  def blk_dma(ib, buf, sem):
    return pltpu.async_copy(
        kmap_hbm.at[pl.ds((base_row + ib * BLKR) * N, BLKR * N)], buf, sem)

  def blk_wait(ib, buf, sem):
    pltpu.make_async_copy(
        kmap_hbm.at[pl.ds((base_row + ib * BLKR) * N, BLKR * N)], buf,
        sem).wait()

  def process(rbuf, blk_row, n):
    def rb(r, n):
      i_g = blk_row + r

      def do_vec(kvec, jv, n):
        mask = kvec >= 0

        def do(n):
          kcl = kvec & 31
          r_in, lastm = plsc.scan_count(kcl, mask)
          rr = r_in - 1
          curv = plsc.load_gather(cur.at[:], [kcl])
          dest = curv + rr
          packed = (kcl << 22) | (i_g << 11) | (jv * LANES + iota)
          n2 = append(n, dest, packed, mask)
          plsc.store_scatter(cur.at[:], [kcl], dest + ones,
                             mask=mask & lastm)
          return n2

        return lax.cond(jnp.any(mask), do, lambda n: n, n)

      def super_body(sv, n):
        off = r * N + sv * (8 * LANES)
        vs = [rbuf[pl.ds(off + u * LANES, LANES)] for u in range(8)]
        m01 = jnp.maximum(vs[0], vs[1])
        m23 = jnp.maximum(vs[2], vs[3])
        m45 = jnp.maximum(vs[4], vs[5])
        m67 = jnp.maximum(vs[6], vs[7])
        m = jnp.maximum(jnp.maximum(m01, m23), jnp.maximum(m45, m67))

        def dosuper(n):
          for u in range(8):
            n = do_vec(vs[u], sv * 8 + u, n)
          return n

        return lax.cond(jnp.any(m >= 0), dosuper, lambda n: n, n)

      return lax.fori_loop(0, N // (8 * LANES), super_body, n)

    return lax.fori_loop(0, BLKR, rb, n)

  blk_dma(0, rowbuf, sem0)

  def pair_body(t, n):
    b0 = 2 * t
    blk_wait(b0, rowbuf, sem0)
    blk_dma(b0 + 1, rowbuf2, sem1)
    n = process(rowbuf, base_row + b0 * BLKR, n)

    def prefetch(x):
      blk_dma(b0 + 2, rowbuf, sem0)
      return x

    lax.cond(t < NBLK // 2 - 1, prefetch, lambda x: x, 0)
    blk_wait(b0 + 1, rowbuf2, sem1)
    return process(rowbuf2, base_row + (b0 + 1) * BLKR, n)

  n = lax.fori_loop(0, NBLK // 2, pair_body, jnp.int32(0))

  nch = lax.shift_right_logical(n + 127, 7)

  def fire(ci, x):
    pltpu.async_copy(lpack.at[ci], packed_hbm.at[didx.at[ci]], dsem)
    return x

  lax.fori_loop(0, nch, fire, 0)

  def drain(ci, x):
    pltpu.make_async_copy(lpack.at[ci], packed_hbm.at[didx.at[ci]],
                          dsem).wait()
    return x

  lax.fori_loop(0, nch, drain, 0)


def _scatter_call(kmap, offs, nev):
  kern = pl.kernel(
      _scatter_kernel,
      out_type=jax.ShapeDtypeStruct((PAD_TOTAL,), jnp.int32),
      mesh=_sc_mesh(),
      compiler_params=_sc_params(),
      scratch_types=[
          pltpu.VMEM((BLKR * N,), jnp.int32),
          pltpu.VMEM((BLKR * N,), jnp.int32),
          pltpu.VMEM((32,), jnp.int32),
          pltpu.VMEM((LANES,), jnp.int32),
          pltpu.VMEM((CAP_ROWS, 128), jnp.int32),
          pltpu.VMEM((CAP_ROWS, 128), jnp.int32),
          pltpu.SemaphoreType.DMA,
          pltpu.SemaphoreType.DMA,
          pltpu.SemaphoreType.DMA,
      ],
  )
  return kern(kmap, offs, nev)


def _expand_kernel(packed_hbm, nev_hbm, cart_t_hbm, shift_t_hbm,
                   src_hbm, dst_hbm, sid_hbm, vx_hbm, vy_hbm, vz_hbm,
                   pbuf, nevbuf, osrc, odst, osid, ovx, ovy, ovz,
                   cx, cy, cz, scx, scy, scz):
  wid = lax.axis_index("s") * 2 + lax.axis_index("c")
  base = wid * SLICE_W
  pltpu.sync_copy(packed_hbm.at[pl.ds(base, SLICE_W)], pbuf)
  pltpu.sync_copy(nev_hbm.at[pl.ds(0, LANES)], nevbuf)
  pltpu.sync_copy(cart_t_hbm.at[pl.ds(0, N)], cx)
  pltpu.sync_copy(cart_t_hbm.at[pl.ds(N, N)], cy)
  pltpu.sync_copy(cart_t_hbm.at[pl.ds(2 * N, N)], cz)
  pltpu.sync_copy(shift_t_hbm.at[pl.ds(0, 32)], scx)
  pltpu.sync_copy(shift_t_hbm.at[pl.ds(32, 32)], scy)
  pltpu.sync_copy(shift_t_hbm.at[pl.ds(64, 32)], scz)

  nev_vec = nevbuf[...]
  iota = lax.iota(jnp.int32, LANES)

  @pl.loop(0, SLICE_W // LANES)
  def _(v):
    sl = pl.ds(v * LANES, LANES)
    gpos = base + v * LANES + iota
    p = jnp.where(gpos < nev_vec, pbuf[sl], 0)
    k = lax.shift_right_logical(p, 22) & 31
    i = lax.shift_right_logical(p, 11) & (N - 1)
    j = p & (N - 1)
    osrc[sl] = i
    odst[sl] = j
    osid[sl] = k
    ovx[sl] = (plsc.load_gather(cx.at[:], [j])
               - plsc.load_gather(cx.at[:], [i])) + plsc.load_gather(
                   scx.at[:], [k])
    ovy[sl] = (plsc.load_gather(cy.at[:], [j])
               - plsc.load_gather(cy.at[:], [i])) + plsc.load_gather(
                   scy.at[:], [k])
    ovz[sl] = (plsc.load_gather(cz.at[:], [j])
               - plsc.load_gather(cz.at[:], [i])) + plsc.load_gather(
                   scz.at[:], [k])

  out_sl = pl.ds(base, SLICE_W)
  pltpu.sync_copy(osrc, src_hbm.at[out_sl])
  pltpu.sync_copy(odst, dst_hbm.at[out_sl])
  pltpu.sync_copy(osid, sid_hbm.at[out_sl])
  pltpu.sync_copy(ovx, vx_hbm.at[out_sl])
  pltpu.sync_copy(ovy, vy_hbm.at[out_sl])
  pltpu.sync_copy(ovz, vz_hbm.at[out_sl])


def _expand_call(packed, nev, cart_t, shift_t):
  kern = pl.kernel(
      _expand_kernel,
      out_type=[jax.ShapeDtypeStruct((PAD_TOTAL,), jnp.int32)] * 3
      + [jax.ShapeDtypeStruct((PAD_TOTAL,), jnp.float32)] * 3,
      mesh=_sc_mesh(),
      compiler_params=_sc_params(),
      scratch_types=[
          pltpu.VMEM((SLICE_W,), jnp.int32),
          pltpu.VMEM((LANES,), jnp.int32),
          pltpu.VMEM((SLICE_W,), jnp.int32),
          pltpu.VMEM((SLICE_W,), jnp.int32),
          pltpu.VMEM((SLICE_W,), jnp.int32),
          pltpu.VMEM((SLICE_W,), jnp.float32),
          pltpu.VMEM((SLICE_W,), jnp.float32),
          pltpu.VMEM((SLICE_W,), jnp.float32),
          pltpu.VMEM((N,), jnp.float32),
          pltpu.VMEM((N,), jnp.float32),
          pltpu.VMEM((N,), jnp.float32),
          pltpu.VMEM((32,), jnp.float32),
          pltpu.VMEM((32,), jnp.float32),
          pltpu.VMEM((32,), jnp.float32),
      ],
  )
  return kern(packed, nev, cart_t, shift_t)


def _dist_body(vx_ref, vy_ref, vz_ref, d_ref):
  vx = vx_ref[...]
  vy = vy_ref[...]
  vz = vz_ref[...]
  q = (vx * vx + vy * vy) + vz * vz
  d_ref[...] = jnp.sqrt(q + 1e-12)


def _dist_call(vx, vy, vz):
  return pl.pallas_call(
      _dist_body,
      out_shape=jax.ShapeDtypeStruct((8, PAD_TOTAL // 8), jnp.float32),
  )(vx, vy, vz)


def kernel(frac_coords, lattice):
  frac = frac_coords.astype(jnp.float32)
  cart = frac @ lattice
  g = np.array([-1.0, 0.0, 1.0])
  shifts = np.stack(np.meshgrid(g, g, g, indexing="ij"), axis=-1).reshape(-1, 3)
  shifts = jnp.asarray(shifts, dtype=jnp.float32)
  shift_cart = shifts @ lattice

  frac_t = frac.T
  cart_t = cart.T
  shift_pad = jnp.concatenate(
      [shift_cart, jnp.zeros((5, 3), jnp.float32)], axis=0)
  shift_t = shift_pad.T

  kmap, counts = _kmap_call(frac, frac_t, cart, cart_t, shift_cart)
  offs, nev = _offsets_call(counts)
  packed = _scatter_call(kmap.reshape(-1), offs.reshape(-1), nev.reshape(-1))
  src, dst, sidx, vx, vy, vz = _expand_call(packed, nev.reshape(-1),
                                            cart_t.reshape(-1),
                                            shift_t.reshape(-1))
  dist = _dist_call(vx.reshape(8, -1), vy.reshape(8, -1),
                    vz.reshape(8, -1)).reshape(-1)
  vec = jnp.stack([vx, vy, vz], axis=-1)
  n_edges = nev[0, 0]
  return (src[:MAX_EDGES], dst[:MAX_EDGES], vec[:MAX_EDGES],
          dist[:MAX_EDGES], n_edges)


# speedup vs baseline: 1.1853x; 1.0842x over previous
import dataclasses
import functools

import jax
import jax.numpy as jnp
import numpy as np
from jax import lax
from jax.experimental import pallas as pl
from jax.experimental.pallas import tpu as pltpu
from jax.experimental.pallas import tpu_sc as plsc

N = 2048
CUT2 = 25.0
EPS2 = 1e-10
MAX_EDGES = 120000
NW = 32
ROWS_W = N // NW
LANES = 16
CAP_ROWS = 96
CAP = CAP_ROWS * 128
TRASH = MAX_EDGES + 448
PAD_TOTAL = 120832
SLICE_W = PAD_TOTAL // NW
BLKR = 8
NBLK = ROWS_W // BLKR
SCAP = 2080


def _kmap_body(fr_ref, ft_ref, cr_ref, ct_ref, sc_ref, kmap_ref, cnt_ref):
  fr = fr_ref[...]
  sct = sc_ref[...]

  frel0 = ft_ref[0:1, :] - fr[:, 0:1]
  frel1 = ft_ref[1:2, :] - fr[:, 1:2]
  frel2 = ft_ref[2:3, :] - fr[:, 2:3]
  s0 = jnp.round(frel0)
  s1 = jnp.round(frel1)
  s2 = jnp.round(frel2)
  kf = (1.0 - s0) * 9.0 + (1.0 - s1) * 3.0 + (1.0 - s2)
  ki = kf.astype(jnp.int32)

  cr = cr_ref[...]
  b0 = ct_ref[0:1, :] - cr[:, 0:1]
  b1 = ct_ref[1:2, :] - cr[:, 1:2]
  b2 = ct_ref[2:3, :] - cr[:, 2:3]

  sc0 = jnp.zeros_like(b0)
  sc1 = jnp.zeros_like(b1)
  sc2 = jnp.zeros_like(b2)
  for k in range(27):
    m = ki == k
    sc0 = jnp.where(m, sct[k, 0], sc0)
    sc1 = jnp.where(m, sct[k, 1], sc1)
    sc2 = jnp.where(m, sct[k, 2], sc2)

  v0 = b0 + sc0
  v1 = b1 + sc1
  v2 = b2 + sc2
  q0 = v0 * v0
  q1 = v1 * v1
  q2 = v2 * v2
  d2 = (q0 + q1) + q2
  edge = (d2 <= CUT2) & (d2 > EPS2)
  kmap_ref[...] = jnp.where(edge, ki, -1)

  kiota = lax.broadcasted_iota(jnp.int32, (1, 1, 32), 2)
  acc = jnp.zeros((1, 1, 32), jnp.int32)
  for k in range(27):
    ck = jnp.sum(jnp.where(edge & (ki == k), 1, 0))
    acc = jnp.where(kiota == k, ck, acc)
  cnt_ref[...] = acc


def _kmap_call(frac, frac_t, cart, cart_t, shift_cart):
  grid = N // ROWS_W
  return pl.pallas_call(
      _kmap_body,
      grid=(grid,),
      in_specs=[
          pl.BlockSpec((ROWS_W, 3), lambda b: (b, 0)),
          pl.BlockSpec((3, N), lambda b: (0, 0)),
          pl.BlockSpec((ROWS_W, 3), lambda b: (b, 0)),
          pl.BlockSpec((3, N), lambda b: (0, 0)),
          pl.BlockSpec((27, 3), lambda b: (0, 0)),
      ],
      out_specs=[
          pl.BlockSpec((ROWS_W, N), lambda b: (b, 0)),
          pl.BlockSpec((1, 1, 32), lambda b: (b, 0, 0)),
      ],
      out_shape=[
          jax.ShapeDtypeStruct((N, N), jnp.int32),
          jax.ShapeDtypeStruct((NW, 1, 32), jnp.int32),
      ],
  )(frac, frac_t, cart, cart_t, shift_cart)


def _incl_cumsum_rows(c, n):
  sh = 1
  while sh < n:
    z = jnp.zeros((sh, c.shape[1]), c.dtype)
    c = c + jnp.concatenate([z, c[: n - sh, :]], axis=0)
    sh *= 2
  return c


def _incl_cumsum_lanes(c, n):
  sh = 1
  while sh < n:
    z = jnp.zeros((c.shape[0], sh), c.dtype)
    c = c + jnp.concatenate([z, c[:, : n - sh]], axis=1)
    sh *= 2
  return c


def _offsets_body(cnt_ref, offs_ref, nev_ref):
  c = cnt_ref[...].reshape(NW, 32)
  colsum = jnp.sum(c, axis=0).reshape(1, 32)
  prefk = _incl_cumsum_lanes(colsum, 32) - colsum
  rowp = _incl_cumsum_rows(c, NW) - c
  offs_ref[...] = prefk + rowp
  total = jnp.sum(colsum)
  nev_ref[...] = jnp.full((8, 128), total, jnp.int32)


def _offsets_call(counts):
  return pl.pallas_call(
      _offsets_body,
      out_shape=[
          jax.ShapeDtypeStruct((NW, 32), jnp.int32),
          jax.ShapeDtypeStruct((8, 128), jnp.int32),
      ],
  )(counts)


def _sc_mesh():
  return plsc.VectorSubcoreMesh(core_axis_name="c", subcore_axis_name="s")


def _sc_params():
  cp = pltpu.CompilerParams()
  if "needs_layout_passes" in pltpu.CompilerParams.__dataclass_fields__:
    cp = dataclasses.replace(cp, needs_layout_passes=False)
  return cp


def _scatter_kernel(kmap_hbm, offs_hbm, nev_hbm, packed_hbm,
                    rowbuf, rowbuf2, cur, nevbuf, stage, didx, lpack,
                    sem0, sem1, dsem):
  wid = lax.axis_index("s") * 2 + lax.axis_index("c")
  base_row = wid * ROWS_W
  iota = lax.iota(jnp.int32, LANES)
  ones = jnp.ones((LANES,), jnp.int32)
  zeros = jnp.zeros((LANES,), jnp.int32)

  pltpu.sync_copy(offs_hbm.at[pl.ds(wid * 32, 32)], cur)
  pltpu.sync_copy(nev_hbm.at[pl.ds(0, LANES)], nevbuf)
  nev_vec = nevbuf[...]

  trash = jnp.full((LANES,), TRASH, jnp.int32)

  @pl.loop(0, CAP_ROWS)
  def _init(r):
    @pl.loop(0, 128 // LANES)
    def _init2(q):
      didx[r, pl.ds(q * LANES, LANES)] = trash

  def append(n, dest, values, mask):
    csm = jnp.cumsum(jnp.where(mask, 1, 0))
    pos = n + csm - 1
    mask = mask & (pos < CAP)
    pr = lax.shift_right_logical(pos, 7)
    pc = pos & 127
    destw = jnp.where(dest < MAX_EDGES, dest, TRASH)
    plsc.store_scatter(didx.at[:], [pr, pc], destw, mask=mask)
    plsc.store_scatter(lpack.at[:], [pr, pc], values, mask=mask)
    return n + jnp.sum(jnp.where(mask, 1, 0))

  del ones

  def blk_dma(ib, buf, sem):
    return pltpu.async_copy(
        kmap_hbm.at[pl.ds((base_row + ib * BLKR) * N, BLKR * N)], buf, sem)

  def blk_wait(ib, buf, sem):
    pltpu.make_async_copy(
        kmap_hbm.at[pl.ds((base_row + ib * BLKR) * N, BLKR * N)], buf,
        sem).wait()

  def process(rbuf, blk_row, n):
    def rb(r, n):
      i_g = blk_row + r
      ibase = i_g << 11

      def vec_body(jv, m_vec):
        kvec = rbuf[pl.ds(r * N + jv * LANES, LANES)]
        mask = kvec >= 0
        kcl = kvec & 31
        packed = (kcl << 22) | ibase | (jv * LANES + iota)
        csm = jnp.cumsum(jnp.where(mask, 1, 0))
        pos = m_vec + csm - 1
        mask = mask & (pos < SCAP - LANES)
        plsc.store_scatter(stage.at[:], [pos], packed, mask=mask)
        return m_vec + plsc.all_reduce_population_count(mask)

      m_vec = lax.fori_loop(0, N // LANES, vec_body, zeros)
      m_s = lax.reduce_max(m_vec, axes=(0,))

      def flush(b, n):
        packed = stage[pl.ds(b * LANES, LANES)]
        fmask = (b * LANES + iota) < m_s
        kcl = lax.shift_right_logical(packed, 22) & 31
        r_in, lastm = plsc.scan_count(kcl, fmask)
        curv = plsc.load_gather(cur.at[:], [kcl])
        dest = curv + r_in - 1
        n2 = append(n, dest, packed, fmask)
        plsc.store_scatter(cur.at[:], [kcl], dest + 1,
                           mask=fmask & lastm)
        return n2

      return lax.fori_loop(0, (m_s + LANES - 1) // LANES, flush, n)

    return lax.fori_loop(0, BLKR, rb, n)

  blk_dma(0, rowbuf, sem0)

  def pair_body(t, n):
    b0 = 2 * t
    blk_wait(b0, rowbuf, sem0)
    blk_dma(b0 + 1, rowbuf2, sem1)
    n = process(rowbuf, base_row + b0 * BLKR, n)

    def prefetch(x):
      blk_dma(b0 + 2, rowbuf, sem0)
      return x

    lax.cond(t < NBLK // 2 - 1, prefetch, lambda x: x, 0)
    blk_wait(b0 + 1, rowbuf2, sem1)
    return process(rowbuf2, base_row + (b0 + 1) * BLKR, n)

  n = lax.fori_loop(0, NBLK // 2, pair_body, jnp.int32(0))

  nch = lax.shift_right_logical(n + 127, 7)

  def fire(ci, x):
    pltpu.async_copy(lpack.at[ci], packed_hbm.at[didx.at[ci]], dsem)
    return x

  lax.fori_loop(0, nch, fire, 0)

  def drain(ci, x):
    pltpu.make_async_copy(lpack.at[ci], packed_hbm.at[didx.at[ci]],
                          dsem).wait()
    return x

  lax.fori_loop(0, nch, drain, 0)


def _scatter_call(kmap, offs, nev):
  kern = pl.kernel(
      _scatter_kernel,
      out_type=jax.ShapeDtypeStruct((PAD_TOTAL,), jnp.int32),
      mesh=_sc_mesh(),
      compiler_params=_sc_params(),
      scratch_types=[
          pltpu.VMEM((BLKR * N,), jnp.int32),
          pltpu.VMEM((BLKR * N,), jnp.int32),
          pltpu.VMEM((32,), jnp.int32),
          pltpu.VMEM((LANES,), jnp.int32),
          pltpu.VMEM((SCAP,), jnp.int32),
          pltpu.VMEM((CAP_ROWS, 128), jnp.int32),
          pltpu.VMEM((CAP_ROWS, 128), jnp.int32),
          pltpu.SemaphoreType.DMA,
          pltpu.SemaphoreType.DMA,
          pltpu.SemaphoreType.DMA,
      ],
  )
  return kern(kmap, offs, nev)


def _expand_kernel(packed_hbm, nev_hbm, cart_t_hbm, shift_t_hbm,
                   src_hbm, dst_hbm, sid_hbm, vx_hbm, vy_hbm, vz_hbm,
                   pbuf, nevbuf, osrc, odst, osid, ovx, ovy, ovz,
                   cx, cy, cz, scx, scy, scz):
  wid = lax.axis_index("s") * 2 + lax.axis_index("c")
  base = wid * SLICE_W
  pltpu.sync_copy(packed_hbm.at[pl.ds(base, SLICE_W)], pbuf)
  pltpu.sync_copy(nev_hbm.at[pl.ds(0, LANES)], nevbuf)
  pltpu.sync_copy(cart_t_hbm.at[pl.ds(0, N)], cx)
  pltpu.sync_copy(cart_t_hbm.at[pl.ds(N, N)], cy)
  pltpu.sync_copy(cart_t_hbm.at[pl.ds(2 * N, N)], cz)
  pltpu.sync_copy(shift_t_hbm.at[pl.ds(0, 32)], scx)
  pltpu.sync_copy(shift_t_hbm.at[pl.ds(32, 32)], scy)
  pltpu.sync_copy(shift_t_hbm.at[pl.ds(64, 32)], scz)

  nev_vec = nevbuf[...]
  iota = lax.iota(jnp.int32, LANES)

  @pl.loop(0, SLICE_W // LANES)
  def _(v):
    sl = pl.ds(v * LANES, LANES)
    gpos = base + v * LANES + iota
    p = jnp.where(gpos < nev_vec, pbuf[sl], 0)
    k = lax.shift_right_logical(p, 22) & 31
    i = lax.shift_right_logical(p, 11) & (N - 1)
    j = p & (N - 1)
    osrc[sl] = i
    odst[sl] = j
    osid[sl] = k
    ovx[sl] = (plsc.load_gather(cx.at[:], [j])
               - plsc.load_gather(cx.at[:], [i])) + plsc.load_gather(
                   scx.at[:], [k])
    ovy[sl] = (plsc.load_gather(cy.at[:], [j])
               - plsc.load_gather(cy.at[:], [i])) + plsc.load_gather(
                   scy.at[:], [k])
    ovz[sl] = (plsc.load_gather(cz.at[:], [j])
               - plsc.load_gather(cz.at[:], [i])) + plsc.load_gather(
                   scz.at[:], [k])

  out_sl = pl.ds(base, SLICE_W)
  pltpu.sync_copy(osrc, src_hbm.at[out_sl])
  pltpu.sync_copy(odst, dst_hbm.at[out_sl])
  pltpu.sync_copy(osid, sid_hbm.at[out_sl])
  pltpu.sync_copy(ovx, vx_hbm.at[out_sl])
  pltpu.sync_copy(ovy, vy_hbm.at[out_sl])
  pltpu.sync_copy(ovz, vz_hbm.at[out_sl])


def _expand_call(packed, nev, cart_t, shift_t):
  kern = pl.kernel(
      _expand_kernel,
      out_type=[jax.ShapeDtypeStruct((PAD_TOTAL,), jnp.int32)] * 3
      + [jax.ShapeDtypeStruct((PAD_TOTAL,), jnp.float32)] * 3,
      mesh=_sc_mesh(),
      compiler_params=_sc_params(),
      scratch_types=[
          pltpu.VMEM((SLICE_W,), jnp.int32),
          pltpu.VMEM((LANES,), jnp.int32),
          pltpu.VMEM((SLICE_W,), jnp.int32),
          pltpu.VMEM((SLICE_W,), jnp.int32),
          pltpu.VMEM((SLICE_W,), jnp.int32),
          pltpu.VMEM((SLICE_W,), jnp.float32),
          pltpu.VMEM((SLICE_W,), jnp.float32),
          pltpu.VMEM((SLICE_W,), jnp.float32),
          pltpu.VMEM((N,), jnp.float32),
          pltpu.VMEM((N,), jnp.float32),
          pltpu.VMEM((N,), jnp.float32),
          pltpu.VMEM((32,), jnp.float32),
          pltpu.VMEM((32,), jnp.float32),
          pltpu.VMEM((32,), jnp.float32),
      ],
  )
  return kern(packed, nev, cart_t, shift_t)


def _dist_body(vx_ref, vy_ref, vz_ref, d_ref):
  vx = vx_ref[...]
  vy = vy_ref[...]
  vz = vz_ref[...]
  q = (vx * vx + vy * vy) + vz * vz
  d_ref[...] = jnp.sqrt(q + 1e-12)


def _dist_call(vx, vy, vz):
  return pl.pallas_call(
      _dist_body,
      out_shape=jax.ShapeDtypeStruct((8, PAD_TOTAL // 8), jnp.float32),
  )(vx, vy, vz)


def kernel(frac_coords, lattice):
  frac = frac_coords.astype(jnp.float32)
  cart = frac @ lattice
  g = np.array([-1.0, 0.0, 1.0])
  shifts = np.stack(np.meshgrid(g, g, g, indexing="ij"), axis=-1).reshape(-1, 3)
  shifts = jnp.asarray(shifts, dtype=jnp.float32)
  shift_cart = shifts @ lattice

  frac_t = frac.T
  cart_t = cart.T
  shift_pad = jnp.concatenate(
      [shift_cart, jnp.zeros((5, 3), jnp.float32)], axis=0)
  shift_t = shift_pad.T

  kmap, counts = _kmap_call(frac, frac_t, cart, cart_t, shift_cart)
  offs, nev = _offsets_call(counts)
  packed = _scatter_call(kmap.reshape(-1), offs.reshape(-1), nev.reshape(-1))
  src, dst, sidx, vx, vy, vz = _expand_call(packed, nev.reshape(-1),
                                            cart_t.reshape(-1),
                                            shift_t.reshape(-1))
  dist = _dist_call(vx.reshape(8, -1), vy.reshape(8, -1),
                    vz.reshape(8, -1)).reshape(-1)
  vec = jnp.stack([vx, vy, vz], axis=-1)
  n_edges = nev[0, 0]
  return (src[:MAX_EDGES], dst[:MAX_EDGES], vec[:MAX_EDGES],
          dist[:MAX_EDGES], n_edges)


# speedup vs baseline: 1.3202x; 1.1138x over previous
import dataclasses
import functools

import jax
import jax.numpy as jnp
import numpy as np
from jax import lax
from jax.experimental import pallas as pl
from jax.experimental.pallas import tpu as pltpu
from jax.experimental.pallas import tpu_sc as plsc

N = 2048
CUT2 = 25.0
EPS2 = 1e-10
MAX_EDGES = 120000
NW = 32
ROWS_W = N // NW
LANES = 16
CAP_ROWS = 96
CAP = CAP_ROWS * 128
TRASH = MAX_EDGES + 448
PAD_TOTAL = 120832
SLICE_W = PAD_TOTAL // NW
BLKR = 8
NBLK = ROWS_W // BLKR
SCAP = 2080


def _kmap_body(fr_ref, ft_ref, cr_ref, ct_ref, sc_ref, kmap_ref, cnt_ref):
  fr = fr_ref[...]
  sct = sc_ref[...]

  frel0 = ft_ref[0:1, :] - fr[:, 0:1]
  frel1 = ft_ref[1:2, :] - fr[:, 1:2]
  frel2 = ft_ref[2:3, :] - fr[:, 2:3]
  s0 = jnp.round(frel0)
  s1 = jnp.round(frel1)
  s2 = jnp.round(frel2)
  kf = (1.0 - s0) * 9.0 + (1.0 - s1) * 3.0 + (1.0 - s2)
  ki = kf.astype(jnp.int32)

  cr = cr_ref[...]
  b0 = ct_ref[0:1, :] - cr[:, 0:1]
  b1 = ct_ref[1:2, :] - cr[:, 1:2]
  b2 = ct_ref[2:3, :] - cr[:, 2:3]

  sc0 = jnp.zeros_like(b0)
  sc1 = jnp.zeros_like(b1)
  sc2 = jnp.zeros_like(b2)
  for k in range(27):
    m = ki == k
    sc0 = jnp.where(m, sct[k, 0], sc0)
    sc1 = jnp.where(m, sct[k, 1], sc1)
    sc2 = jnp.where(m, sct[k, 2], sc2)

  v0 = b0 + sc0
  v1 = b1 + sc1
  v2 = b2 + sc2
  q0 = v0 * v0
  q1 = v1 * v1
  q2 = v2 * v2
  d2 = (q0 + q1) + q2
  edge = (d2 <= CUT2) & (d2 > EPS2)
  kmap_ref[...] = jnp.where(edge, ki, -1)

  kiota = lax.broadcasted_iota(jnp.int32, (1, 1, 32), 2)
  acc = jnp.zeros((1, 1, 32), jnp.int32)
  for k in range(27):
    ck = jnp.sum(jnp.where(edge & (ki == k), 1, 0))
    acc = jnp.where(kiota == k, ck, acc)
  cnt_ref[...] = acc


def _kmap_call(frac, frac_t, cart, cart_t, shift_cart):
  grid = N // ROWS_W
  return pl.pallas_call(
      _kmap_body,
      grid=(grid,),
      in_specs=[
          pl.BlockSpec((ROWS_W, 3), lambda b: (b, 0)),
          pl.BlockSpec((3, N), lambda b: (0, 0)),
          pl.BlockSpec((ROWS_W, 3), lambda b: (b, 0)),
          pl.BlockSpec((3, N), lambda b: (0, 0)),
          pl.BlockSpec((27, 3), lambda b: (0, 0)),
      ],
      out_specs=[
          pl.BlockSpec((ROWS_W, N), lambda b: (b, 0)),
          pl.BlockSpec((1, 1, 32), lambda b: (b, 0, 0)),
      ],
      out_shape=[
          jax.ShapeDtypeStruct((N, N), jnp.int32),
          jax.ShapeDtypeStruct((NW, 1, 32), jnp.int32),
      ],
  )(frac, frac_t, cart, cart_t, shift_cart)


def _incl_cumsum_rows(c, n):
  sh = 1
  while sh < n:
    z = jnp.zeros((sh, c.shape[1]), c.dtype)
    c = c + jnp.concatenate([z, c[: n - sh, :]], axis=0)
    sh *= 2
  return c


def _incl_cumsum_lanes(c, n):
  sh = 1
  while sh < n:
    z = jnp.zeros((c.shape[0], sh), c.dtype)
    c = c + jnp.concatenate([z, c[:, : n - sh]], axis=1)
    sh *= 2
  return c


def _offsets_body(cnt_ref, offs_ref, nev_ref):
  c = cnt_ref[...].reshape(NW, 32)
  colsum = jnp.sum(c, axis=0).reshape(1, 32)
  prefk = _incl_cumsum_lanes(colsum, 32) - colsum
  rowp = _incl_cumsum_rows(c, NW) - c
  offs_ref[...] = prefk + rowp
  total = jnp.sum(colsum)
  nev_ref[...] = jnp.full((8, 128), total, jnp.int32)


def _offsets_call(counts):
  return pl.pallas_call(
      _offsets_body,
      out_shape=[
          jax.ShapeDtypeStruct((NW, 32), jnp.int32),
          jax.ShapeDtypeStruct((8, 128), jnp.int32),
      ],
  )(counts)


def _sc_mesh():
  return plsc.VectorSubcoreMesh(core_axis_name="c", subcore_axis_name="s")


def _sc_params():
  cp = pltpu.CompilerParams()
  if "needs_layout_passes" in pltpu.CompilerParams.__dataclass_fields__:
    cp = dataclasses.replace(cp, needs_layout_passes=False)
  return cp


def _scatter_kernel(kmap_hbm, offs_hbm, nev_hbm, packed_hbm,
                    rowbuf, rowbuf2, cur, nevbuf, stage, didx, lpack,
                    sem0, sem1, dsem):
  wid = lax.axis_index("s") * 2 + lax.axis_index("c")
  base_row = wid * ROWS_W
  iota = lax.iota(jnp.int32, LANES)
  ones = jnp.ones((LANES,), jnp.int32)
  zeros = jnp.zeros((LANES,), jnp.int32)

  pltpu.sync_copy(offs_hbm.at[pl.ds(wid * 32, 32)], cur)
  pltpu.sync_copy(nev_hbm.at[pl.ds(0, LANES)], nevbuf)
  nev_vec = nevbuf[...]

  trash = jnp.full((LANES,), TRASH, jnp.int32)

  @pl.loop(0, CAP_ROWS)
  def _init(r):
    @pl.loop(0, 128 // LANES)
    def _init2(q):
      didx[r, pl.ds(q * LANES, LANES)] = trash

  def append(n, dest, values, mask):
    csm = jnp.cumsum(jnp.where(mask, 1, 0))
    pos = n + csm - 1
    mask = mask & (pos < CAP)
    pr = lax.shift_right_logical(pos, 7)
    pc = pos & 127
    destw = jnp.where(dest < MAX_EDGES, dest, TRASH)
    plsc.store_scatter(didx.at[:], [pr, pc], destw, mask=mask)
    plsc.store_scatter(lpack.at[:], [pr, pc], values, mask=mask)
    return n + jnp.sum(jnp.where(mask, 1, 0))

  del ones

  def blk_dma(ib, buf, sem):
    return pltpu.async_copy(
        kmap_hbm.at[pl.ds((base_row + ib * BLKR) * N, BLKR * N)], buf, sem)

  def blk_wait(ib, buf, sem):
    pltpu.make_async_copy(
        kmap_hbm.at[pl.ds((base_row + ib * BLKR) * N, BLKR * N)], buf,
        sem).wait()

  def process(rbuf, blk_row, n):
    def rb(r, n):
      i_g = blk_row + r
      ibase = i_g << 11

      @plsc.parallel_loop(0, N // LANES, unroll=8, carry=zeros)
      def m_vec(jv, m_vec):
        kvec = rbuf[pl.ds(r * N + jv * LANES, LANES)]
        mask = kvec >= 0
        kcl = kvec & 31
        packed = (kcl << 22) | ibase | (jv * LANES + iota)
        csm = jnp.cumsum(jnp.where(mask, 1, 0))
        pos = m_vec + csm - 1
        mask = mask & (pos < SCAP - LANES)
        plsc.store_scatter(stage.at[:], [pos], packed, mask=mask)
        return m_vec + plsc.all_reduce_population_count(mask)
      m_s = lax.reduce_max(m_vec, axes=(0,))

      def flush(b, n):
        packed = stage[pl.ds(b * LANES, LANES)]
        fmask = (b * LANES + iota) < m_s
        kcl = lax.shift_right_logical(packed, 22) & 31
        r_in, lastm = plsc.scan_count(kcl, fmask)
        curv = plsc.load_gather(cur.at[:], [kcl])
        dest = curv + r_in - 1
        n2 = append(n, dest, packed, fmask)
        plsc.store_scatter(cur.at[:], [kcl], dest + 1,
                           mask=fmask & lastm)
        return n2

      return lax.fori_loop(0, (m_s + LANES - 1) // LANES, flush, n)

    return lax.fori_loop(0, BLKR, rb, n)

  blk_dma(0, rowbuf, sem0)

  def pair_body(t, n):
    b0 = 2 * t
    blk_wait(b0, rowbuf, sem0)
    blk_dma(b0 + 1, rowbuf2, sem1)
    n = process(rowbuf, base_row + b0 * BLKR, n)

    def prefetch(x):
      blk_dma(b0 + 2, rowbuf, sem0)
      return x

    lax.cond(t < NBLK // 2 - 1, prefetch, lambda x: x, 0)
    blk_wait(b0 + 1, rowbuf2, sem1)
    return process(rowbuf2, base_row + (b0 + 1) * BLKR, n)

  n = lax.fori_loop(0, NBLK // 2, pair_body, jnp.int32(0))

  nch = lax.shift_right_logical(n + 127, 7)

  def fire(ci, x):
    pltpu.async_copy(lpack.at[ci], packed_hbm.at[didx.at[ci]], dsem)
    return x

  lax.fori_loop(0, nch, fire, 0)

  def drain(ci, x):
    pltpu.make_async_copy(lpack.at[ci], packed_hbm.at[didx.at[ci]],
                          dsem).wait()
    return x

  lax.fori_loop(0, nch, drain, 0)


def _scatter_call(kmap, offs, nev):
  kern = pl.kernel(
      _scatter_kernel,
      out_type=jax.ShapeDtypeStruct((PAD_TOTAL,), jnp.int32),
      mesh=_sc_mesh(),
      compiler_params=_sc_params(),
      scratch_types=[
          pltpu.VMEM((BLKR * N,), jnp.int32),
          pltpu.VMEM((BLKR * N,), jnp.int32),
          pltpu.VMEM((32,), jnp.int32),
          pltpu.VMEM((LANES,), jnp.int32),
          pltpu.VMEM((SCAP,), jnp.int32),
          pltpu.VMEM((CAP_ROWS, 128), jnp.int32),
          pltpu.VMEM((CAP_ROWS, 128), jnp.int32),
          pltpu.SemaphoreType.DMA,
          pltpu.SemaphoreType.DMA,
          pltpu.SemaphoreType.DMA,
      ],
  )
  return kern(kmap, offs, nev)


def _expand_kernel(packed_hbm, nev_hbm, cart_t_hbm, shift_t_hbm,
                   src_hbm, dst_hbm, sid_hbm, vx_hbm, vy_hbm, vz_hbm,
                   pbuf, nevbuf, osrc, odst, osid, ovx, ovy, ovz,
                   cx, cy, cz, scx, scy, scz):
  wid = lax.axis_index("s") * 2 + lax.axis_index("c")
  base = wid * SLICE_W
  pltpu.sync_copy(packed_hbm.at[pl.ds(base, SLICE_W)], pbuf)
  pltpu.sync_copy(nev_hbm.at[pl.ds(0, LANES)], nevbuf)
  pltpu.sync_copy(cart_t_hbm.at[pl.ds(0, N)], cx)
  pltpu.sync_copy(cart_t_hbm.at[pl.ds(N, N)], cy)
  pltpu.sync_copy(cart_t_hbm.at[pl.ds(2 * N, N)], cz)
  pltpu.sync_copy(shift_t_hbm.at[pl.ds(0, 32)], scx)
  pltpu.sync_copy(shift_t_hbm.at[pl.ds(32, 32)], scy)
  pltpu.sync_copy(shift_t_hbm.at[pl.ds(64, 32)], scz)

  nev_vec = nevbuf[...]
  iota = lax.iota(jnp.int32, LANES)

  @pl.loop(0, SLICE_W // LANES)
  def _(v):
    sl = pl.ds(v * LANES, LANES)
    gpos = base + v * LANES + iota
    p = jnp.where(gpos < nev_vec, pbuf[sl], 0)
    k = lax.shift_right_logical(p, 22) & 31
    i = lax.shift_right_logical(p, 11) & (N - 1)
    j = p & (N - 1)
    osrc[sl] = i
    odst[sl] = j
    osid[sl] = k
    ovx[sl] = (plsc.load_gather(cx.at[:], [j])
               - plsc.load_gather(cx.at[:], [i])) + plsc.load_gather(
                   scx.at[:], [k])
    ovy[sl] = (plsc.load_gather(cy.at[:], [j])
               - plsc.load_gather(cy.at[:], [i])) + plsc.load_gather(
                   scy.at[:], [k])
    ovz[sl] = (plsc.load_gather(cz.at[:], [j])
               - plsc.load_gather(cz.at[:], [i])) + plsc.load_gather(
                   scz.at[:], [k])

  out_sl = pl.ds(base, SLICE_W)
  pltpu.sync_copy(osrc, src_hbm.at[out_sl])
  pltpu.sync_copy(odst, dst_hbm.at[out_sl])
  pltpu.sync_copy(osid, sid_hbm.at[out_sl])
  pltpu.sync_copy(ovx, vx_hbm.at[out_sl])
  pltpu.sync_copy(ovy, vy_hbm.at[out_sl])
  pltpu.sync_copy(ovz, vz_hbm.at[out_sl])


def _expand_call(packed, nev, cart_t, shift_t):
  kern = pl.kernel(
      _expand_kernel,
      out_type=[jax.ShapeDtypeStruct((PAD_TOTAL,), jnp.int32)] * 3
      + [jax.ShapeDtypeStruct((PAD_TOTAL,), jnp.float32)] * 3,
      mesh=_sc_mesh(),
      compiler_params=_sc_params(),
      scratch_types=[
          pltpu.VMEM((SLICE_W,), jnp.int32),
          pltpu.VMEM((LANES,), jnp.int32),
          pltpu.VMEM((SLICE_W,), jnp.int32),
          pltpu.VMEM((SLICE_W,), jnp.int32),
          pltpu.VMEM((SLICE_W,), jnp.int32),
          pltpu.VMEM((SLICE_W,), jnp.float32),
          pltpu.VMEM((SLICE_W,), jnp.float32),
          pltpu.VMEM((SLICE_W,), jnp.float32),
          pltpu.VMEM((N,), jnp.float32),
          pltpu.VMEM((N,), jnp.float32),
          pltpu.VMEM((N,), jnp.float32),
          pltpu.VMEM((32,), jnp.float32),
          pltpu.VMEM((32,), jnp.float32),
          pltpu.VMEM((32,), jnp.float32),
      ],
  )
  return kern(packed, nev, cart_t, shift_t)


def _dist_body(vx_ref, vy_ref, vz_ref, d_ref):
  vx = vx_ref[...]
  vy = vy_ref[...]
  vz = vz_ref[...]
  q = (vx * vx + vy * vy) + vz * vz
  d_ref[...] = jnp.sqrt(q + 1e-12)


def _dist_call(vx, vy, vz):
  return pl.pallas_call(
      _dist_body,
      out_shape=jax.ShapeDtypeStruct((8, PAD_TOTAL // 8), jnp.float32),
  )(vx, vy, vz)


def kernel(frac_coords, lattice):
  frac = frac_coords.astype(jnp.float32)
  cart = frac @ lattice
  g = np.array([-1.0, 0.0, 1.0])
  shifts = np.stack(np.meshgrid(g, g, g, indexing="ij"), axis=-1).reshape(-1, 3)
  shifts = jnp.asarray(shifts, dtype=jnp.float32)
  shift_cart = shifts @ lattice

  frac_t = frac.T
  cart_t = cart.T
  shift_pad = jnp.concatenate(
      [shift_cart, jnp.zeros((5, 3), jnp.float32)], axis=0)
  shift_t = shift_pad.T

  kmap, counts = _kmap_call(frac, frac_t, cart, cart_t, shift_cart)
  offs, nev = _offsets_call(counts)
  packed = _scatter_call(kmap.reshape(-1), offs.reshape(-1), nev.reshape(-1))
  src, dst, sidx, vx, vy, vz = _expand_call(packed, nev.reshape(-1),
                                            cart_t.reshape(-1),
                                            shift_t.reshape(-1))
  dist = _dist_call(vx.reshape(8, -1), vy.reshape(8, -1),
                    vz.reshape(8, -1)).reshape(-1)
  vec = jnp.stack([vx, vy, vz], axis=-1)
  n_edges = nev[0, 0]
  return (src[:MAX_EDGES], dst[:MAX_EDGES], vec[:MAX_EDGES],
          dist[:MAX_EDGES], n_edges)


# speedup vs baseline: 1.5910x; 1.2051x over previous
import dataclasses
import functools

import jax
import jax.numpy as jnp
import numpy as np
from jax import lax
from jax.experimental import pallas as pl
from jax.experimental.pallas import tpu as pltpu
from jax.experimental.pallas import tpu_sc as plsc

N = 2048
CUT2 = 25.0
EPS2 = 1e-10
MAX_EDGES = 120000
NW = 32
ROWS_W = N // NW
LANES = 16
CAP_ROWS = 96
CAP = CAP_ROWS * 128
TRASH = MAX_EDGES + 448
PAD_TOTAL = 120832
SLICE_W = PAD_TOTAL // NW
BLKR = 8
NBLK = ROWS_W // BLKR
SCAP = 2080


def _kmap_body(fr_ref, ft_ref, cr_ref, ct_ref, sc_ref, kmap_ref, cnt_ref):
  fr = fr_ref[...]
  sct = sc_ref[...]

  frel0 = ft_ref[0:1, :] - fr[:, 0:1]
  frel1 = ft_ref[1:2, :] - fr[:, 1:2]
  frel2 = ft_ref[2:3, :] - fr[:, 2:3]
  s0 = jnp.round(frel0)
  s1 = jnp.round(frel1)
  s2 = jnp.round(frel2)
  kf = (1.0 - s0) * 9.0 + (1.0 - s1) * 3.0 + (1.0 - s2)
  ki = kf.astype(jnp.int32)

  cr = cr_ref[...]
  b0 = ct_ref[0:1, :] - cr[:, 0:1]
  b1 = ct_ref[1:2, :] - cr[:, 1:2]
  b2 = ct_ref[2:3, :] - cr[:, 2:3]

  def lane_take(col):
    tab = jnp.broadcast_to(col.reshape(1, 27), (ROWS_W, 27))
    return jnp.take_along_axis(tab, ki, axis=1)

  sc0 = lane_take(sct[:, 0])
  sc1 = lane_take(sct[:, 1])
  sc2 = lane_take(sct[:, 2])

  v0 = b0 + sc0
  v1 = b1 + sc1
  v2 = b2 + sc2
  q0 = v0 * v0
  q1 = v1 * v1
  q2 = v2 * v2
  d2 = (q0 + q1) + q2
  edge = (d2 <= CUT2) & (d2 > EPS2)
  kidx = jnp.where(edge, ki, -1)
  kmap_ref[...] = kidx

  kiota = lax.broadcasted_iota(jnp.int32, (1, 1, 32), 2)
  acc = jnp.zeros((1, 1, 32), jnp.int32)
  for k in range(27):
    ck = jnp.sum(jnp.where(kidx == k, 1, 0))
    acc = jnp.where(kiota == k, ck, acc)
  cnt_ref[...] = acc


def _kmap_call(frac, frac_t, cart, cart_t, shift_cart):
  grid = N // ROWS_W
  return pl.pallas_call(
      _kmap_body,
      grid=(grid,),
      in_specs=[
          pl.BlockSpec((ROWS_W, 3), lambda b: (b, 0)),
          pl.BlockSpec((3, N), lambda b: (0, 0)),
          pl.BlockSpec((ROWS_W, 3), lambda b: (b, 0)),
          pl.BlockSpec((3, N), lambda b: (0, 0)),
          pl.BlockSpec((27, 3), lambda b: (0, 0)),
      ],
      out_specs=[
          pl.BlockSpec((ROWS_W, N), lambda b: (b, 0)),
          pl.BlockSpec((1, 1, 32), lambda b: (b, 0, 0)),
      ],
      out_shape=[
          jax.ShapeDtypeStruct((N, N), jnp.int32),
          jax.ShapeDtypeStruct((NW, 1, 32), jnp.int32),
      ],
  )(frac, frac_t, cart, cart_t, shift_cart)


def _incl_cumsum_rows(c, n):
  sh = 1
  while sh < n:
    z = jnp.zeros((sh, c.shape[1]), c.dtype)
    c = c + jnp.concatenate([z, c[: n - sh, :]], axis=0)
    sh *= 2
  return c


def _incl_cumsum_lanes(c, n):
  sh = 1
  while sh < n:
    z = jnp.zeros((c.shape[0], sh), c.dtype)
    c = c + jnp.concatenate([z, c[:, : n - sh]], axis=1)
    sh *= 2
  return c


def _offsets_body(cnt_ref, offs_ref, nev_ref):
  c = cnt_ref[...].reshape(NW, 32)
  colsum = jnp.sum(c, axis=0).reshape(1, 32)
  prefk = _incl_cumsum_lanes(colsum, 32) - colsum
  rowp = _incl_cumsum_rows(c, NW) - c
  offs_ref[...] = prefk + rowp
  total = jnp.sum(colsum)
  nev_ref[...] = jnp.full((8, 128), total, jnp.int32)


def _offsets_call(counts):
  return pl.pallas_call(
      _offsets_body,
      out_shape=[
          jax.ShapeDtypeStruct((NW, 32), jnp.int32),
          jax.ShapeDtypeStruct((8, 128), jnp.int32),
      ],
  )(counts)


def _sc_mesh():
  return plsc.VectorSubcoreMesh(core_axis_name="c", subcore_axis_name="s")


def _sc_params():
  cp = pltpu.CompilerParams()
  if "needs_layout_passes" in pltpu.CompilerParams.__dataclass_fields__:
    cp = dataclasses.replace(cp, needs_layout_passes=False)
  return cp


def _scatter_kernel(kmap_hbm, offs_hbm, nev_hbm, packed_hbm,
                    rowbuf, rowbuf2, cur, nevbuf, stage, didx, lpack,
                    sem0, sem1, dsem):
  wid = lax.axis_index("s") * 2 + lax.axis_index("c")
  base_row = wid * ROWS_W
  iota = lax.iota(jnp.int32, LANES)
  ones = jnp.ones((LANES,), jnp.int32)
  zeros = jnp.zeros((LANES,), jnp.int32)

  pltpu.sync_copy(offs_hbm.at[pl.ds(wid * 32, 32)], cur)
  pltpu.sync_copy(nev_hbm.at[pl.ds(0, LANES)], nevbuf)
  nev_vec = nevbuf[...]

  trash = jnp.full((LANES,), TRASH, jnp.int32)

  @pl.loop(0, CAP_ROWS)
  def _init(r):
    @pl.loop(0, 128 // LANES)
    def _init2(q):
      didx[r, pl.ds(q * LANES, LANES)] = trash

  def append(n, dest, values, mask):
    csm = jnp.cumsum(jnp.where(mask, 1, 0))
    pos = n + csm - 1
    mask = mask & (pos < CAP)
    pr = lax.shift_right_logical(pos, 7)
    pc = pos & 127
    destw = jnp.where(dest < MAX_EDGES, dest, TRASH)
    plsc.store_scatter(didx.at[:], [pr, pc], destw, mask=mask)
    plsc.store_scatter(lpack.at[:], [pr, pc], values, mask=mask)
    return n + jnp.sum(jnp.where(mask, 1, 0))

  del ones

  def blk_dma(ib, buf, sem):
    return pltpu.async_copy(
        kmap_hbm.at[pl.ds((base_row + ib * BLKR) * N, BLKR * N)], buf, sem)

  def blk_wait(ib, buf, sem):
    pltpu.make_async_copy(
        kmap_hbm.at[pl.ds((base_row + ib * BLKR) * N, BLKR * N)], buf,
        sem).wait()

  def process(rbuf, blk_row, n):
    def rb(r, n):
      i_g = blk_row + r
      ibase = i_g << 11

      @plsc.parallel_loop(0, N // LANES, unroll=8, carry=zeros)
      def m_vec(jv, m_vec):
        kvec = rbuf[pl.ds(r * N + jv * LANES, LANES)]
        mask = kvec >= 0
        kcl = kvec & 31
        packed = (kcl << 22) | ibase | (jv * LANES + iota)
        csm = jnp.cumsum(jnp.where(mask, 1, 0))
        pos = m_vec + csm - 1
        mask = mask & (pos < SCAP - LANES)
        plsc.store_scatter(stage.at[:], [pos], packed, mask=mask)
        return m_vec + plsc.all_reduce_population_count(mask)
      m_s = lax.reduce_max(m_vec, axes=(0,))

      def flush(b, n):
        packed = stage[pl.ds(b * LANES, LANES)]
        fmask = (b * LANES + iota) < m_s
        kcl = lax.shift_right_logical(packed, 22) & 31
        r_in, lastm = plsc.scan_count(kcl, fmask)
        curv = plsc.load_gather(cur.at[:], [kcl])
        dest = curv + r_in - 1
        n2 = append(n, dest, packed, fmask)
        plsc.store_scatter(cur.at[:], [kcl], dest + 1,
                           mask=fmask & lastm)
        return n2

      return lax.fori_loop(0, (m_s + LANES - 1) // LANES, flush, n)

    return lax.fori_loop(0, BLKR, rb, n)

  blk_dma(0, rowbuf, sem0)

  def pair_body(t, n):
    b0 = 2 * t
    blk_wait(b0, rowbuf, sem0)
    blk_dma(b0 + 1, rowbuf2, sem1)
    n = process(rowbuf, base_row + b0 * BLKR, n)

    def prefetch(x):
      blk_dma(b0 + 2, rowbuf, sem0)
      return x

    lax.cond(t < NBLK // 2 - 1, prefetch, lambda x: x, 0)
    blk_wait(b0 + 1, rowbuf2, sem1)
    return process(rowbuf2, base_row + (b0 + 1) * BLKR, n)

  n = lax.fori_loop(0, NBLK // 2, pair_body, jnp.int32(0))

  nch = lax.shift_right_logical(n + 127, 7)

  def fire(ci, x):
    pltpu.async_copy(lpack.at[ci], packed_hbm.at[didx.at[ci]], dsem)
    return x

  lax.fori_loop(0, nch, fire, 0)

  def drain(ci, x):
    pltpu.make_async_copy(lpack.at[ci], packed_hbm.at[didx.at[ci]],
                          dsem).wait()
    return x

  lax.fori_loop(0, nch, drain, 0)


def _scatter_call(kmap, offs, nev):
  kern = pl.kernel(
      _scatter_kernel,
      out_type=jax.ShapeDtypeStruct((PAD_TOTAL,), jnp.int32),
      mesh=_sc_mesh(),
      compiler_params=_sc_params(),
      scratch_types=[
          pltpu.VMEM((BLKR * N,), jnp.int32),
          pltpu.VMEM((BLKR * N,), jnp.int32),
          pltpu.VMEM((32,), jnp.int32),
          pltpu.VMEM((LANES,), jnp.int32),
          pltpu.VMEM((SCAP,), jnp.int32),
          pltpu.VMEM((CAP_ROWS, 128), jnp.int32),
          pltpu.VMEM((CAP_ROWS, 128), jnp.int32),
          pltpu.SemaphoreType.DMA,
          pltpu.SemaphoreType.DMA,
          pltpu.SemaphoreType.DMA,
      ],
  )
  return kern(kmap, offs, nev)


def _expand_kernel(packed_hbm, nev_hbm, cart_t_hbm, shift_t_hbm,
                   src_hbm, dst_hbm, sid_hbm, vx_hbm, vy_hbm, vz_hbm,
                   pbuf, nevbuf, osrc, odst, osid, ovx, ovy, ovz,
                   cx, cy, cz, scx, scy, scz):
  wid = lax.axis_index("s") * 2 + lax.axis_index("c")
  base = wid * SLICE_W
  pltpu.sync_copy(packed_hbm.at[pl.ds(base, SLICE_W)], pbuf)
  pltpu.sync_copy(nev_hbm.at[pl.ds(0, LANES)], nevbuf)
  pltpu.sync_copy(cart_t_hbm.at[pl.ds(0, N)], cx)
  pltpu.sync_copy(cart_t_hbm.at[pl.ds(N, N)], cy)
  pltpu.sync_copy(cart_t_hbm.at[pl.ds(2 * N, N)], cz)
  pltpu.sync_copy(shift_t_hbm.at[pl.ds(0, 32)], scx)
  pltpu.sync_copy(shift_t_hbm.at[pl.ds(32, 32)], scy)
  pltpu.sync_copy(shift_t_hbm.at[pl.ds(64, 32)], scz)

  nev_vec = nevbuf[...]
  iota = lax.iota(jnp.int32, LANES)

  @pl.loop(0, SLICE_W // LANES)
  def _(v):
    sl = pl.ds(v * LANES, LANES)
    gpos = base + v * LANES + iota
    p = jnp.where(gpos < nev_vec, pbuf[sl], 0)
    k = lax.shift_right_logical(p, 22) & 31
    i = lax.shift_right_logical(p, 11) & (N - 1)
    j = p & (N - 1)
    osrc[sl] = i
    odst[sl] = j
    osid[sl] = k
    ovx[sl] = (plsc.load_gather(cx.at[:], [j])
               - plsc.load_gather(cx.at[:], [i])) + plsc.load_gather(
                   scx.at[:], [k])
    ovy[sl] = (plsc.load_gather(cy.at[:], [j])
               - plsc.load_gather(cy.at[:], [i])) + plsc.load_gather(
                   scy.at[:], [k])
    ovz[sl] = (plsc.load_gather(cz.at[:], [j])
               - plsc.load_gather(cz.at[:], [i])) + plsc.load_gather(
                   scz.at[:], [k])

  out_sl = pl.ds(base, SLICE_W)
  pltpu.sync_copy(osrc, src_hbm.at[out_sl])
  pltpu.sync_copy(odst, dst_hbm.at[out_sl])
  pltpu.sync_copy(osid, sid_hbm.at[out_sl])
  pltpu.sync_copy(ovx, vx_hbm.at[out_sl])
  pltpu.sync_copy(ovy, vy_hbm.at[out_sl])
  pltpu.sync_copy(ovz, vz_hbm.at[out_sl])


def _expand_call(packed, nev, cart_t, shift_t):
  kern = pl.kernel(
      _expand_kernel,
      out_type=[jax.ShapeDtypeStruct((PAD_TOTAL,), jnp.int32)] * 3
      + [jax.ShapeDtypeStruct((PAD_TOTAL,), jnp.float32)] * 3,
      mesh=_sc_mesh(),
      compiler_params=_sc_params(),
      scratch_types=[
          pltpu.VMEM((SLICE_W,), jnp.int32),
          pltpu.VMEM((LANES,), jnp.int32),
          pltpu.VMEM((SLICE_W,), jnp.int32),
          pltpu.VMEM((SLICE_W,), jnp.int32),
          pltpu.VMEM((SLICE_W,), jnp.int32),
          pltpu.VMEM((SLICE_W,), jnp.float32),
          pltpu.VMEM((SLICE_W,), jnp.float32),
          pltpu.VMEM((SLICE_W,), jnp.float32),
          pltpu.VMEM((N,), jnp.float32),
          pltpu.VMEM((N,), jnp.float32),
          pltpu.VMEM((N,), jnp.float32),
          pltpu.VMEM((32,), jnp.float32),
          pltpu.VMEM((32,), jnp.float32),
          pltpu.VMEM((32,), jnp.float32),
      ],
  )
  return kern(packed, nev, cart_t, shift_t)


def _dist_body(vx_ref, vy_ref, vz_ref, d_ref):
  vx = vx_ref[...]
  vy = vy_ref[...]
  vz = vz_ref[...]
  q = (vx * vx + vy * vy) + vz * vz
  d_ref[...] = jnp.sqrt(q + 1e-12)


def _dist_call(vx, vy, vz):
  return pl.pallas_call(
      _dist_body,
      out_shape=jax.ShapeDtypeStruct((8, PAD_TOTAL // 8), jnp.float32),
  )(vx, vy, vz)


def kernel(frac_coords, lattice):
  frac = frac_coords.astype(jnp.float32)
  cart = frac @ lattice
  g = np.array([-1.0, 0.0, 1.0])
  shifts = np.stack(np.meshgrid(g, g, g, indexing="ij"), axis=-1).reshape(-1, 3)
  shifts = jnp.asarray(shifts, dtype=jnp.float32)
  shift_cart = shifts @ lattice

  frac_t = frac.T
  cart_t = cart.T
  shift_pad = jnp.concatenate(
      [shift_cart, jnp.zeros((5, 3), jnp.float32)], axis=0)
  shift_t = shift_pad.T

  kmap, counts = _kmap_call(frac, frac_t, cart, cart_t, shift_cart)
  offs, nev = _offsets_call(counts)
  packed = _scatter_call(kmap.reshape(-1), offs.reshape(-1), nev.reshape(-1))
  src, dst, sidx, vx, vy, vz = _expand_call(packed, nev.reshape(-1),
                                            cart_t.reshape(-1),
                                            shift_t.reshape(-1))
  dist = _dist_call(vx.reshape(8, -1), vy.reshape(8, -1),
                    vz.reshape(8, -1)).reshape(-1)
  vec = jnp.stack([vx, vy, vz], axis=-1)
  n_edges = nev[0, 0]
  return (src[:MAX_EDGES], dst[:MAX_EDGES], vec[:MAX_EDGES],
          dist[:MAX_EDGES], n_edges)


# speedup vs baseline: 1.5917x; 1.0004x over previous
import dataclasses

import jax
import jax.numpy as jnp
import numpy as np
from jax import lax
from jax.experimental import pallas as pl
from jax.experimental.pallas import tpu as pltpu
from jax.experimental.pallas import tpu_sc as plsc

N = 2048
CUT2 = 25.0
EPS2 = 1e-10
MAX_EDGES = 120000
NW = 32
ROWS_W = N // NW
LANES = 16
CAP_ROWS = 96
CAP = CAP_ROWS * 128
TRASH = MAX_EDGES + 448
PAD_TOTAL = 120832
SLICE_W = PAD_TOTAL // NW
BLKR = 8
NBLK = ROWS_W // BLKR
SCAP = 2080


def _kmap_body(fr_ref, ft_ref, cr_ref, ct_ref, sc_ref, kmap_ref, cnt_ref):
  fr = fr_ref[...]
  sct = sc_ref[...]

  frel0 = ft_ref[0:1, :] - fr[:, 0:1]
  frel1 = ft_ref[1:2, :] - fr[:, 1:2]
  frel2 = ft_ref[2:3, :] - fr[:, 2:3]
  s0 = jnp.round(frel0)
  s1 = jnp.round(frel1)
  s2 = jnp.round(frel2)
  kf = (1.0 - s0) * 9.0 + (1.0 - s1) * 3.0 + (1.0 - s2)
  ki = kf.astype(jnp.int32)

  cr = cr_ref[...]
  b0 = ct_ref[0:1, :] - cr[:, 0:1]
  b1 = ct_ref[1:2, :] - cr[:, 1:2]
  b2 = ct_ref[2:3, :] - cr[:, 2:3]

  def lane_take(col):
    tab = jnp.broadcast_to(col.reshape(1, 27), (ROWS_W, 27))
    return jnp.take_along_axis(tab, ki, axis=1)

  sc0 = lane_take(sct[:, 0])
  sc1 = lane_take(sct[:, 1])
  sc2 = lane_take(sct[:, 2])

  v0 = b0 + sc0
  v1 = b1 + sc1
  v2 = b2 + sc2
  q0 = v0 * v0
  q1 = v1 * v1
  q2 = v2 * v2
  d2 = (q0 + q1) + q2
  edge = (d2 <= CUT2) & (d2 > EPS2)
  kidx = jnp.where(edge, ki, -1)
  kmap_ref[...] = kidx

  kiota = lax.broadcasted_iota(jnp.int32, (1, 1, 32), 2)
  acc = jnp.zeros((1, 1, 32), jnp.int32)
  for k in range(27):
    ck = jnp.sum(jnp.where(kidx == k, 1, 0))
    acc = jnp.where(kiota == k, ck, acc)
  cnt_ref[...] = acc


def _kmap_call(frac, frac_t, cart, cart_t, shift_cart):
  grid = N // ROWS_W
  return pl.pallas_call(
      _kmap_body,
      grid=(grid,),
      in_specs=[
          pl.BlockSpec((ROWS_W, 3), lambda b: (b, 0)),
          pl.BlockSpec((3, N), lambda b: (0, 0)),
          pl.BlockSpec((ROWS_W, 3), lambda b: (b, 0)),
          pl.BlockSpec((3, N), lambda b: (0, 0)),
          pl.BlockSpec((27, 3), lambda b: (0, 0)),
      ],
      out_specs=[
          pl.BlockSpec((ROWS_W, N), lambda b: (b, 0)),
          pl.BlockSpec((1, 1, 32), lambda b: (b, 0, 0)),
      ],
      out_shape=[
          jax.ShapeDtypeStruct((N, N), jnp.int32),
          jax.ShapeDtypeStruct((NW, 1, 32), jnp.int32),
      ],
  )(frac, frac_t, cart, cart_t, shift_cart)


def _incl_cumsum_rows(c, n):
  sh = 1
  while sh < n:
    z = jnp.zeros((sh, c.shape[1]), c.dtype)
    c = c + jnp.concatenate([z, c[: n - sh, :]], axis=0)
    sh *= 2
  return c


def _incl_cumsum_lanes(c, n):
  sh = 1
  while sh < n:
    z = jnp.zeros((c.shape[0], sh), c.dtype)
    c = c + jnp.concatenate([z, c[:, : n - sh]], axis=1)
    sh *= 2
  return c


def _offsets_body(cnt_ref, offs_ref, nev_ref):
  c = cnt_ref[...].reshape(NW, 32)
  colsum = jnp.sum(c, axis=0).reshape(1, 32)
  prefk = _incl_cumsum_lanes(colsum, 32) - colsum
  rowp = _incl_cumsum_rows(c, NW) - c
  offs_ref[...] = prefk + rowp
  total = jnp.sum(colsum)
  nev_ref[...] = jnp.full((8, 128), total, jnp.int32)


def _offsets_call(counts):
  return pl.pallas_call(
      _offsets_body,
      out_shape=[
          jax.ShapeDtypeStruct((NW, 32), jnp.int32),
          jax.ShapeDtypeStruct((8, 128), jnp.int32),
      ],
  )(counts)


def _sc_mesh():
  return plsc.VectorSubcoreMesh(core_axis_name="c", subcore_axis_name="s")


def _sc_params():
  cp = pltpu.CompilerParams()
  if "needs_layout_passes" in pltpu.CompilerParams.__dataclass_fields__:
    cp = dataclasses.replace(cp, needs_layout_passes=False)
  return cp


def _scatter_kernel(kmap_hbm, offs_hbm, nev_hbm, packed_hbm,
                    rowbuf, rowbuf2, cur, nevbuf, stage, didx, lpack,
                    sem0, sem1, dsem):
  wid = lax.axis_index("s") * 2 + lax.axis_index("c")
  base_row = wid * ROWS_W
  iota = lax.iota(jnp.int32, LANES)
  ones = jnp.ones((LANES,), jnp.int32)
  zeros = jnp.zeros((LANES,), jnp.int32)

  pltpu.sync_copy(offs_hbm.at[pl.ds(wid * 32, 32)], cur)
  pltpu.sync_copy(nev_hbm.at[pl.ds(0, LANES)], nevbuf)

  trash = jnp.full((LANES,), TRASH, jnp.int32)

  @pl.loop(0, CAP_ROWS)
  def _init(r):
    @pl.loop(0, 128 // LANES)
    def _init2(q):
      didx[r, pl.ds(q * LANES, LANES)] = trash

  def append(n, dest, values, mask):
    csm = jnp.cumsum(jnp.where(mask, 1, 0))
    pos = n + csm - 1
    mask = mask & (pos < CAP)
    pr = lax.shift_right_logical(pos, 7)
    pc = pos & 127
    destw = jnp.where(dest < MAX_EDGES, dest, TRASH)
    plsc.store_scatter(didx.at[:], [pr, pc], destw, mask=mask)
    plsc.store_scatter(lpack.at[:], [pr, pc], values, mask=mask)
    return n + jnp.sum(jnp.where(mask, 1, 0))

  del ones

  def blk_dma(ib, buf, sem):
    return pltpu.async_copy(
        kmap_hbm.at[pl.ds((base_row + ib * BLKR) * N, BLKR * N)], buf, sem)

  def blk_wait(ib, buf, sem):
    pltpu.make_async_copy(
        kmap_hbm.at[pl.ds((base_row + ib * BLKR) * N, BLKR * N)], buf,
        sem).wait()

  def process(rbuf, blk_row, n):
    def rb(r, n):
      i_g = blk_row + r
      ibase = i_g << 11

      @plsc.parallel_loop(0, N // LANES, unroll=8, carry=zeros)
      def m_vec(jv, m_vec):
        kvec = rbuf[pl.ds(r * N + jv * LANES, LANES)]
        mask = kvec >= 0
        kcl = kvec & 31
        packed = (kcl << 22) | ibase | (jv * LANES + iota)
        csm = jnp.cumsum(jnp.where(mask, 1, 0))
        pos = m_vec + csm - 1
        mask = mask & (pos < SCAP - LANES)
        plsc.store_scatter(stage.at[:], [pos], packed, mask=mask)
        return m_vec + plsc.all_reduce_population_count(mask)
      m_s = lax.reduce_max(m_vec, axes=(0,))

      def flush(b, n):
        packed = stage[pl.ds(b * LANES, LANES)]
        fmask = (b * LANES + iota) < m_s
        kcl = lax.shift_right_logical(packed, 22) & 31
        r_in, lastm = plsc.scan_count(kcl, fmask)
        curv = plsc.load_gather(cur.at[:], [kcl])
        dest = curv + r_in - 1
        n2 = append(n, dest, packed, fmask)
        plsc.store_scatter(cur.at[:], [kcl], dest + 1,
                           mask=fmask & lastm)
        return n2

      return lax.fori_loop(0, (m_s + LANES - 1) // LANES, flush, n)

    return lax.fori_loop(0, BLKR, rb, n)

  blk_dma(0, rowbuf, sem0)

  def pair_body(t, n):
    b0 = 2 * t
    blk_wait(b0, rowbuf, sem0)
    blk_dma(b0 + 1, rowbuf2, sem1)
    n = process(rowbuf, base_row + b0 * BLKR, n)

    def prefetch(x):
      blk_dma(b0 + 2, rowbuf, sem0)
      return x

    lax.cond(t < NBLK // 2 - 1, prefetch, lambda x: x, 0)
    blk_wait(b0 + 1, rowbuf2, sem1)
    return process(rowbuf2, base_row + (b0 + 1) * BLKR, n)

  n = lax.fori_loop(0, NBLK // 2, pair_body, jnp.int32(0))

  nch = lax.shift_right_logical(n + 127, 7)

  def fire(ci, x):
    pltpu.async_copy(lpack.at[ci], packed_hbm.at[didx.at[ci]], dsem)
    return x

  lax.fori_loop(0, nch, fire, 0)

  def drain(ci, x):
    pltpu.make_async_copy(lpack.at[ci], packed_hbm.at[didx.at[ci]],
                          dsem).wait()
    return x

  lax.fori_loop(0, nch, drain, 0)


def _scatter_call(kmap, offs, nev):
  kern = pl.kernel(
      _scatter_kernel,
      out_type=jax.ShapeDtypeStruct((PAD_TOTAL,), jnp.int32),
      mesh=_sc_mesh(),
      compiler_params=_sc_params(),
      scratch_types=[
          pltpu.VMEM((BLKR * N,), jnp.int32),
          pltpu.VMEM((BLKR * N,), jnp.int32),
          pltpu.VMEM((32,), jnp.int32),
          pltpu.VMEM((LANES,), jnp.int32),
          pltpu.VMEM((SCAP,), jnp.int32),
          pltpu.VMEM((CAP_ROWS, 128), jnp.int32),
          pltpu.VMEM((CAP_ROWS, 128), jnp.int32),
          pltpu.SemaphoreType.DMA,
          pltpu.SemaphoreType.DMA,
          pltpu.SemaphoreType.DMA,
      ],
  )
  return kern(kmap, offs, nev)


def _expand_kernel(packed_hbm, nev_hbm, cart_t_hbm, shift_t_hbm,
                   src_hbm, dst_hbm, sid_hbm, vx_hbm, vy_hbm, vz_hbm,
                   pbuf, nevbuf, osrc, odst, osid, ovx, ovy, ovz,
                   cx, cy, cz, scx, scy, scz):
  wid = lax.axis_index("s") * 2 + lax.axis_index("c")
  base = wid * SLICE_W
  pltpu.sync_copy(packed_hbm.at[pl.ds(base, SLICE_W)], pbuf)
  pltpu.sync_copy(nev_hbm.at[pl.ds(0, LANES)], nevbuf)
  pltpu.sync_copy(cart_t_hbm.at[pl.ds(0, N)], cx)
  pltpu.sync_copy(cart_t_hbm.at[pl.ds(N, N)], cy)
  pltpu.sync_copy(cart_t_hbm.at[pl.ds(2 * N, N)], cz)
  pltpu.sync_copy(shift_t_hbm.at[pl.ds(0, 32)], scx)
  pltpu.sync_copy(shift_t_hbm.at[pl.ds(32, 32)], scy)
  pltpu.sync_copy(shift_t_hbm.at[pl.ds(64, 32)], scz)

  nev_vec = nevbuf[...]
  iota = lax.iota(jnp.int32, LANES)

  @pl.loop(0, SLICE_W // LANES)
  def _(v):
    sl = pl.ds(v * LANES, LANES)
    gpos = base + v * LANES + iota
    p = jnp.where(gpos < nev_vec, pbuf[sl], 0)
    k = lax.shift_right_logical(p, 22) & 31
    i = lax.shift_right_logical(p, 11) & (N - 1)
    j = p & (N - 1)
    osrc[sl] = i
    odst[sl] = j
    osid[sl] = k
    ovx[sl] = (plsc.load_gather(cx.at[:], [j])
               - plsc.load_gather(cx.at[:], [i])) + plsc.load_gather(
                   scx.at[:], [k])
    ovy[sl] = (plsc.load_gather(cy.at[:], [j])
               - plsc.load_gather(cy.at[:], [i])) + plsc.load_gather(
                   scy.at[:], [k])
    ovz[sl] = (plsc.load_gather(cz.at[:], [j])
               - plsc.load_gather(cz.at[:], [i])) + plsc.load_gather(
                   scz.at[:], [k])

  out_sl = pl.ds(base, SLICE_W)
  pltpu.sync_copy(osrc, src_hbm.at[out_sl])
  pltpu.sync_copy(odst, dst_hbm.at[out_sl])
  pltpu.sync_copy(osid, sid_hbm.at[out_sl])
  pltpu.sync_copy(ovx, vx_hbm.at[out_sl])
  pltpu.sync_copy(ovy, vy_hbm.at[out_sl])
  pltpu.sync_copy(ovz, vz_hbm.at[out_sl])


def _expand_call(packed, nev, cart_t, shift_t):
  kern = pl.kernel(
      _expand_kernel,
      out_type=[jax.ShapeDtypeStruct((PAD_TOTAL,), jnp.int32)] * 3
      + [jax.ShapeDtypeStruct((PAD_TOTAL,), jnp.float32)] * 3,
      mesh=_sc_mesh(),
      compiler_params=_sc_params(),
      scratch_types=[
          pltpu.VMEM((SLICE_W,), jnp.int32),
          pltpu.VMEM((LANES,), jnp.int32),
          pltpu.VMEM((SLICE_W,), jnp.int32),
          pltpu.VMEM((SLICE_W,), jnp.int32),
          pltpu.VMEM((SLICE_W,), jnp.int32),
          pltpu.VMEM((SLICE_W,), jnp.float32),
          pltpu.VMEM((SLICE_W,), jnp.float32),
          pltpu.VMEM((SLICE_W,), jnp.float32),
          pltpu.VMEM((N,), jnp.float32),
          pltpu.VMEM((N,), jnp.float32),
          pltpu.VMEM((N,), jnp.float32),
          pltpu.VMEM((32,), jnp.float32),
          pltpu.VMEM((32,), jnp.float32),
          pltpu.VMEM((32,), jnp.float32),
      ],
  )
  return kern(packed, nev, cart_t, shift_t)


def _dist_body(vx_ref, vy_ref, vz_ref, d_ref):
  vx = vx_ref[...]
  vy = vy_ref[...]
  vz = vz_ref[...]
  q = (vx * vx + vy * vy) + vz * vz
  d_ref[...] = jnp.sqrt(q + 1e-12)


def _dist_call(vx, vy, vz):
  return pl.pallas_call(
      _dist_body,
      out_shape=jax.ShapeDtypeStruct((8, PAD_TOTAL // 8), jnp.float32),
  )(vx, vy, vz)


def kernel(frac_coords, lattice):
  frac = frac_coords.astype(jnp.float32)
  cart = frac @ lattice
  g = np.array([-1.0, 0.0, 1.0])
  shifts = np.stack(np.meshgrid(g, g, g, indexing="ij"), axis=-1).reshape(-1, 3)
  shifts = jnp.asarray(shifts, dtype=jnp.float32)
  shift_cart = shifts @ lattice

  frac_t = frac.T
  cart_t = cart.T
  shift_pad = jnp.concatenate(
      [shift_cart, jnp.zeros((5, 3), jnp.float32)], axis=0)
  shift_t = shift_pad.T

  kmap, counts = _kmap_call(frac, frac_t, cart, cart_t, shift_cart)
  offs, nev = _offsets_call(counts)
  packed = _scatter_call(kmap.reshape(-1), offs.reshape(-1), nev.reshape(-1))
  src, dst, sidx, vx, vy, vz = _expand_call(packed, nev.reshape(-1),
                                            cart_t.reshape(-1),
                                            shift_t.reshape(-1))
  dist = _dist_call(vx.reshape(8, -1), vy.reshape(8, -1),
                    vz.reshape(8, -1)).reshape(-1)
  vec = jnp.stack([vx, vy, vz], axis=-1)
  n_edges = nev[0, 0]
  return (src[:MAX_EDGES], dst[:MAX_EDGES], vec[:MAX_EDGES],
          dist[:MAX_EDGES], n_edges)
